# baseline (speedup 1.0000x reference)
import functools

import jax
import jax.numpy as jnp
from jax import lax
from jax.experimental import pallas as pl
from jax.experimental.pallas import tpu as pltpu
from jax.experimental.pallas import tpu_sc as plsc

NC = 2
NS = 16
L = 16
NW = NC * NS

EMBED_DIM = 32
CHUNK = 128


def _rsqrt_nr(x):
    xi = lax.bitcast_convert_type(x, jnp.int32)
    yi = jnp.int32(0x5F3759DF) - lax.shift_right_arithmetic(xi, jnp.int32(1))
    y = lax.bitcast_convert_type(yi, jnp.float32)
    for _ in range(3):
        y = y * (1.5 - 0.5 * x * y * y)
    return y


def _make_sc_kernel(batch):
    b_per_w = batch // NW
    n_chunks = b_per_w // CHUNK
    n_groups = b_per_w // L
    mesh = plsc.VectorSubcoreMesh(
        core_axis_name="c", subcore_axis_name="s", num_cores=NC, num_subcores=NS
    )

    @functools.partial(
        pl.kernel,
        out_type=jax.ShapeDtypeStruct((batch,), jnp.float32),
        mesh=mesh,
        scratch_types=[
            pltpu.VMEM((n_chunks, CHUNK), jnp.int32),
            pltpu.VMEM((n_chunks, CHUNK), jnp.int32),
            pltpu.VMEM((b_per_w, EMBED_DIM), jnp.float32),
            pltpu.VMEM((b_per_w, EMBED_DIM), jnp.float32),
            pltpu.VMEM((b_per_w,), jnp.float32),
            pltpu.SemaphoreType.DMA,
        ],
        compiler_params=pltpu.CompilerParams(
            needs_layout_passes=False, use_tc_tiling_on_sc=False
        ),
    )
    def sc_kernel(utab_hbm, ntab_hbm, uids_hbm, nids_hbm, out_hbm,
                  uidx_v, nidx_v, urows_v, nrows_v, scores_v, sem):
        wid = lax.axis_index("s") * NC + lax.axis_index("c")
        base = wid * b_per_w

        pltpu.sync_copy(uids_hbm.at[wid], uidx_v)
        pltpu.sync_copy(nids_hbm.at[wid], nidx_v)

        copies = []
        for j in range(n_chunks):
            dst = pl.ds(j * CHUNK, CHUNK)
            copies.append(pltpu.async_copy(utab_hbm.at[uidx_v.at[j]], urows_v.at[dst], sem))
            copies.append(pltpu.async_copy(ntab_hbm.at[nidx_v.at[j]], nrows_v.at[dst], sem))
        for c in copies:
            c.wait()

        iota16 = lax.iota(jnp.int32, L)
        zero = jnp.zeros((L,), jnp.float32)
        tiny = jnp.full((L,), 1e-36, jnp.float32)
        cap = jnp.full((L,), 1e12, jnp.float32)

        def g_body(g, _):
            rows = g * L + iota16

            def d_body(d, carry):
                un, uu, nn = carry
                cols = jnp.full((L,), d, jnp.int32)
                u = plsc.load_gather(urows_v, [rows, cols])
                n = plsc.load_gather(nrows_v, [rows, cols])
                return (un + u * n, uu + u * u, nn + n * n)

            un, uu, nn = lax.fori_loop(0, EMBED_DIM, d_body, (zero, zero, zero))
            inv_u = jnp.minimum(_rsqrt_nr(jnp.maximum(uu, tiny)), cap)
            inv_n = jnp.minimum(_rsqrt_nr(jnp.maximum(nn, tiny)), cap)
            scores_v[pl.ds(g * L, L)] = un * inv_u * inv_n
            return 0

        lax.fori_loop(0, n_groups, g_body, 0)
        pltpu.sync_copy(scores_v, out_hbm.at[pl.ds(base, b_per_w)])

    return sc_kernel


def kernel(user_ids, nonprofit_ids, user_table, nonprofit_table):
    batch = user_ids.shape[0]
    uids3 = user_ids.astype(jnp.int32).reshape(NW, batch // NW // CHUNK, CHUNK)
    nids3 = nonprofit_ids.astype(jnp.int32).reshape(NW, batch // NW // CHUNK, CHUNK)
    return _make_sc_kernel(batch)(user_table, nonprofit_table, uids3, nids3)
